# Initial kernel scaffold; baseline (speedup 1.0000x reference)
import jax
import jax.numpy as jnp
from jax.experimental import pallas as pl


def kernel(x, edge_index, edge_attr, W1, b1, W2, b2, Wl, bl):
    raise NotImplementedError("write your pallas kernel here")


# speedup vs baseline: 30.7401x; 30.7401x over previous
import functools

import numpy as np

import jax
import jax.numpy as jnp
from jax import lax
from jax.experimental import pallas as pl
from jax.experimental.pallas import tpu as pltpu
from jax.experimental.pallas import tpu_sc as plsc
from jax._src.config import enable_x64 as _x64_ctx

_N = 10000
_E = 320000
_D = 128
_K = 5

_NC = 2
_NS = 16
_NW = _NC * _NS
_CH = 128
_NCH = 80
_EP = _NW * _NCH * _CH
_NP = 10240
_TROWS = _NP // _NS

_f32 = jnp.float32
_i32 = jnp.int32


def _iota16():
    return lax.iota(_i32, 16)


def _splat(v):
    if v.dtype != _i32:
        v = lax.convert_element_type(v, _i32)
    return lax.broadcast(v, (16,))


def _fori(hi, body):
    @pl.loop(np.int32(0), np.int32(hi), step=np.int32(1))
    def _(i):
        body(i, np.int32(0))


def _sc_deg_body(row_t, ea_t, degp, acc, rowbuf, eabuf, sbuf):
    cid = lax.axis_index("c")
    sid = lax.axis_index("s")
    wid = cid * np.int32(_NS) + sid
    iota = _iota16()
    zero16 = jnp.zeros((16,), _f32)

    def _zb(i, c):
        fi = _splat(i)
        for s in range(_D // 16):
            plsc.store_scatter(sbuf, [fi, iota + np.int32(16 * s)], zero16)
        return c
    _fori(_CH, _zb)

    def _zc(b, c):
        pltpu.sync_copy(sbuf, acc.at[pl.ds(sid * np.int32(_TROWS) + b * np.int32(_CH), _CH)])
        return c
    _fori(_TROWS // _CH, _zc)
    plsc.subcore_barrier()

    pltpu.sync_copy(row_t.at[wid], rowbuf)
    pltpu.sync_copy(ea_t.at[wid], eabuf)

    def _chunk(j, c):
        fj = _splat(j)

        def _edge(e, c2):
            fe = _splat(e)
            a = plsc.load_gather(eabuf, [fj, fe])
            for s in range(_D // 16):
                plsc.store_scatter(sbuf, [fe, iota + np.int32(16 * s)], a)
            return c2
        _fori(_CH, _edge)
        pltpu.sync_copy(sbuf, acc.at[rowbuf.at[j]], add=True)
        return c
    _fori(_NCH, _chunk)

    plsc.subcore_barrier()
    pltpu.sync_copy(acc.at[pl.ds(sid * np.int32(_TROWS), _TROWS)],
                    degp.at[cid, pl.ds(sid * np.int32(_TROWS), _TROWS)])


_sc_deg = pl.kernel(
    _sc_deg_body,
    out_type=jax.ShapeDtypeStruct((_NC, _NP, _D), _f32),
    mesh=plsc.VectorSubcoreMesh(core_axis_name="c", subcore_axis_name="s"),
    compiler_params=pltpu.CompilerParams(needs_layout_passes=False),
    scratch_types=[
        pltpu.VMEM_SHARED((_NP, _D), _f32),
        pltpu.VMEM((_NCH, _CH), _i32),
        pltpu.VMEM((_NCH, _CH), _f32),
        pltpu.VMEM((_CH, _D), _f32),
    ],
)


def _sc_spmv_body(u, row_t, col_t, ea_t, p_out,
                  acc, rowbuf, colbuf, eabuf, rows_v, sem):
    cid = lax.axis_index("c")
    sid = lax.axis_index("s")
    wid = cid * np.int32(_NS) + sid
    iota = _iota16()
    zero16 = jnp.zeros((16,), _f32)

    def _zb(i, c):
        fi = _splat(i)
        for s in range(_D // 16):
            plsc.store_scatter(rows_v, [fi, iota + np.int32(16 * s)], zero16)
        return c
    _fori(_CH, _zb)

    def _zc(b, c):
        pltpu.sync_copy(rows_v, acc.at[pl.ds(sid * np.int32(_TROWS) + b * np.int32(_CH), _CH)])
        return c
    _fori(_TROWS // _CH, _zc)
    plsc.subcore_barrier()

    pltpu.sync_copy(row_t.at[wid], rowbuf)
    pltpu.sync_copy(col_t.at[wid], colbuf)
    pltpu.sync_copy(ea_t.at[wid], eabuf)

    def _chunk(j, c):
        fj = _splat(j)
        pltpu.async_copy(u.at[rowbuf.at[j]], rows_v, sem).wait()

        def _edge(e, c2):
            fe = _splat(e)
            a = -plsc.load_gather(eabuf, [fj, fe])
            for s in range(_D // 16):
                ix = iota + np.int32(16 * s)
                v = plsc.load_gather(rows_v, [fe, ix])
                plsc.store_scatter(rows_v, [fe, ix], v * a)
            return c2
        _fori(_CH, _edge)
        pltpu.sync_copy(rows_v, acc.at[colbuf.at[j]], add=True)
        return c
    _fori(_NCH, _chunk)

    plsc.subcore_barrier()
    pltpu.sync_copy(acc.at[pl.ds(sid * np.int32(_TROWS), _TROWS)],
                    p_out.at[cid, pl.ds(sid * np.int32(_TROWS), _TROWS)])


_sc_spmv = pl.kernel(
    _sc_spmv_body,
    out_type=jax.ShapeDtypeStruct((_NC, _NP, _D), _f32),
    mesh=plsc.VectorSubcoreMesh(core_axis_name="c", subcore_axis_name="s"),
    compiler_params=pltpu.CompilerParams(needs_layout_passes=False),
    scratch_types=[
        pltpu.VMEM_SHARED((_NP, _D), _f32),
        pltpu.VMEM((_NCH, _CH), _i32),
        pltpu.VMEM((_NCH, _CH), _i32),
        pltpu.VMEM((_NCH, _CH), _f32),
        pltpu.VMEM((_CH, _D), _f32),
        pltpu.SemaphoreType.DMA,
    ],
)


def _gelu(v):
    return 0.5 * v * (1.0 + lax.erf(v * 0.7071067811865476))


def _tc_prep_body(degp_ref, x_ref, dinv_ref, u0_ref):
    d = degp_ref[0, :, 0:1] + degp_ref[1, :, 0:1]
    safe = jnp.where(d > 0, d, 1.0)
    dinv = jnp.where(d > 0, 1.0 / jnp.sqrt(safe), 0.0)
    db = jnp.broadcast_to(dinv, (_NP, _D))
    dinv_ref[...] = db
    u0_ref[...] = db * x_ref[...]


_tc_prep = pl.pallas_call(
    _tc_prep_body,
    out_shape=(jax.ShapeDtypeStruct((_NP, _D), _f32),
               jax.ShapeDtypeStruct((_NP, _D), _f32)),
)


def _tc_step1_body(p_ref, dinv_ref, tx_ref, u_ref):
    db = dinv_ref[...]
    t = db * (p_ref[0] + p_ref[1])
    tx_ref[...] = t
    u_ref[...] = db * t


_tc_step1 = pl.pallas_call(
    _tc_step1_body,
    out_shape=(jax.ShapeDtypeStruct((_NP, _D), _f32),
               jax.ShapeDtypeStruct((_NP, _D), _f32)),
)


def _tc_stepk_body(p_ref, dinv_ref, tprev_ref, tx_ref, u_ref):
    db = dinv_ref[...]
    t = 2.0 * (db * (p_ref[0] + p_ref[1])) - tprev_ref[...]
    tx_ref[...] = t
    u_ref[...] = db * t


_tc_stepk = pl.pallas_call(
    _tc_stepk_body,
    out_shape=(jax.ShapeDtypeStruct((_NP, _D), _f32),
               jax.ShapeDtypeStruct((_NP, _D), _f32)),
)


def _tc_accum1_body(v0_ref, t1_ref, t2_ref, t3_ref, t4_ref, w_ref, b_ref,
                    dinv_ref, h_ref, uh_ref):
    acc = jnp.dot(v0_ref[...], w_ref[0], preferred_element_type=_f32)
    acc += jnp.dot(t1_ref[...], w_ref[1], preferred_element_type=_f32)
    acc += jnp.dot(t2_ref[...], w_ref[2], preferred_element_type=_f32)
    acc += jnp.dot(t3_ref[...], w_ref[3], preferred_element_type=_f32)
    acc += jnp.dot(t4_ref[...], w_ref[4], preferred_element_type=_f32)
    h = _gelu(acc + b_ref[...])
    h_ref[...] = h
    uh_ref[...] = dinv_ref[...] * h


_tc_accum1 = pl.pallas_call(
    _tc_accum1_body,
    out_shape=(jax.ShapeDtypeStruct((_NP, _D), _f32),
               jax.ShapeDtypeStruct((_NP, _D), _f32)),
)


def _tc_accum2_body(v0_ref, t1_ref, t2_ref, t3_ref, t4_ref, w_ref, b_ref,
                    x_ref, wlt_ref, bl_ref, out_ref):
    acc = jnp.dot(v0_ref[...], w_ref[0], preferred_element_type=_f32)
    acc += jnp.dot(t1_ref[...], w_ref[1], preferred_element_type=_f32)
    acc += jnp.dot(t2_ref[...], w_ref[2], preferred_element_type=_f32)
    acc += jnp.dot(t3_ref[...], w_ref[3], preferred_element_type=_f32)
    acc += jnp.dot(t4_ref[...], w_ref[4], preferred_element_type=_f32)
    lin = jnp.dot(x_ref[...], wlt_ref[...], preferred_element_type=_f32)
    out_ref[...] = _gelu(acc + b_ref[...] + lin + bl_ref[...])


_tc_accum2 = pl.pallas_call(
    _tc_accum2_body,
    out_shape=jax.ShapeDtypeStruct((_NP, _D), _f32),
)


def kernel(x, edge_index, edge_attr, W1, b1, W2, b2, Wl, bl):
    with _x64_ctx(False):
        out = _kernel32(x, edge_index, edge_attr, W1, b1, W2, b2, Wl, bl)
    return out.astype(jnp.float64)


def _kernel32(x, edge_index, edge_attr, W1, b1, W2, b2, Wl, bl):
    x = x.astype(_f32)
    row = edge_index[0].astype(_i32)
    col = edge_index[1].astype(_i32)
    ea = edge_attr.astype(_f32)
    W1 = W1.astype(_f32)
    W2 = W2.astype(_f32)

    pad = _EP - _E
    row_t = jnp.concatenate([row, jnp.zeros((pad,), _i32)]).reshape(_NW, _NCH, _CH)
    col_t = jnp.concatenate([col, jnp.zeros((pad,), _i32)]).reshape(_NW, _NCH, _CH)
    ea_t = jnp.concatenate([ea, jnp.zeros((pad,), _f32)]).reshape(_NW, _NCH, _CH)
    x_pad = jnp.concatenate([x, jnp.zeros((_NP - _N, _D), _f32)], axis=0)

    b1r = b1.astype(_f32).reshape(1, _D)
    b2r = b2.astype(_f32).reshape(1, _D)
    blr = bl.astype(_f32).reshape(1, _D)
    wlt = Wl.astype(_f32).T

    degp = _sc_deg(row_t, ea_t)
    dinv_b, u0 = _tc_prep(degp, x_pad)

    def cheb_txs(v0, u_first):
        p = _sc_spmv(u_first, row_t, col_t, ea_t)
        t1, u = _tc_step1(p, dinv_b)
        p = _sc_spmv(u, row_t, col_t, ea_t)
        t2, u = _tc_stepk(p, dinv_b, v0)
        p = _sc_spmv(u, row_t, col_t, ea_t)
        t3, u = _tc_stepk(p, dinv_b, t1)
        p = _sc_spmv(u, row_t, col_t, ea_t)
        t4, _ = _tc_stepk(p, dinv_b, t2)
        return t1, t2, t3, t4

    t1, t2, t3, t4 = cheb_txs(x_pad, u0)
    h, uh = _tc_accum1(x_pad, t1, t2, t3, t4, W1, b1r, dinv_b)

    t1, t2, t3, t4 = cheb_txs(h, uh)
    out_pad = _tc_accum2(h, t1, t2, t3, t4, W2, b2r, x_pad, wlt, blr)

    return out_pad[:_N]


# speedup vs baseline: 53.4824x; 1.7398x over previous
import functools

import numpy as np

import jax
import jax.numpy as jnp
from jax import lax
from jax.experimental import pallas as pl
from jax.experimental.pallas import tpu as pltpu
from jax.experimental.pallas import tpu_sc as plsc
from jax._src.config import enable_x64 as _x64_ctx

_N = 10000
_E = 320000
_D = 128
_K = 5

_NC = 2
_NS = 16
_NW = _NC * _NS
_CH = 128
_NCH = 80
_EP = _NW * _NCH * _CH
_NP = 10240
_TROWS = _NP // _NS

_f32 = jnp.float32
_i32 = jnp.int32


def _iota16():
    return lax.iota(_i32, 16)


def _splat(v):
    if v.dtype != _i32:
        v = lax.convert_element_type(v, _i32)
    return lax.broadcast(v, (16,))


def _fori(hi, body):
    @pl.loop(np.int32(0), np.int32(hi), step=np.int32(1))
    def _(i):
        body(i, np.int32(0))


def _sc_deg_body(row_t, ea_t, degp, acc, rowbuf, eabuf, sbuf):
    cid = lax.axis_index("c")
    sid = lax.axis_index("s")
    wid = cid * np.int32(_NS) + sid
    iota = _iota16()
    zero16 = jnp.zeros((16,), _f32)

    @pl.loop(np.int32(0), np.int32(_CH), step=np.int32(1), unroll=8)
    def _zb(i):
        fi = _splat(i)
        for s in range(_D // 16):
            plsc.store_scatter(sbuf, [fi, iota + np.int32(16 * s)], zero16)

    def _zc(b, c):
        pltpu.sync_copy(sbuf, acc.at[pl.ds(sid * np.int32(_TROWS) + b * np.int32(_CH), _CH)])
        return c
    _fori(_TROWS // _CH, _zc)
    plsc.subcore_barrier()

    pltpu.sync_copy(row_t.at[wid], rowbuf)
    pltpu.sync_copy(ea_t.at[wid], eabuf)

    def _chunk(j, c):
        fj = _splat(j)

        @pl.loop(np.int32(0), np.int32(_CH), step=np.int32(1), unroll=8)
        def _edge(e):
            fe = _splat(e)
            a = plsc.load_gather(eabuf, [fj, fe])
            for s in range(_D // 16):
                plsc.store_scatter(sbuf, [fe, iota + np.int32(16 * s)], a)
        pltpu.sync_copy(sbuf, acc.at[rowbuf.at[j]], add=True)
        return c
    _fori(_NCH, _chunk)

    plsc.subcore_barrier()
    pltpu.sync_copy(acc.at[pl.ds(sid * np.int32(_TROWS), _TROWS)],
                    degp.at[cid, pl.ds(sid * np.int32(_TROWS), _TROWS)])


_sc_deg = pl.kernel(
    _sc_deg_body,
    out_type=jax.ShapeDtypeStruct((_NC, _NP, _D), _f32),
    mesh=plsc.VectorSubcoreMesh(core_axis_name="c", subcore_axis_name="s"),
    compiler_params=pltpu.CompilerParams(needs_layout_passes=False),
    scratch_types=[
        pltpu.VMEM_SHARED((_NP, _D), _f32),
        pltpu.VMEM((_NCH, _CH), _i32),
        pltpu.VMEM((_NCH, _CH), _f32),
        pltpu.VMEM((_CH, _D), _f32),
    ],
)


def _sc_spmv_body(u, row_t, col_t, ea_t, p_out,
                  acc, rib, cib, eabuf, rows_a, rows_b,
                  gsem_a, gsem_b, isem_a, isem_b):
    cid = lax.axis_index("c")
    sid = lax.axis_index("s")
    wid = cid * np.int32(_NS) + sid
    iota = _iota16()
    zero16 = jnp.zeros((16,), _f32)
    bufs = (rows_a, rows_b)
    gsems = (gsem_a, gsem_b)
    isems = (isem_a, isem_b)

    @pl.loop(np.int32(0), np.int32(_CH), step=np.int32(1), unroll=8)
    def _zb(i):
        fi = _splat(i)
        for s in range(_D // 16):
            plsc.store_scatter(rows_a, [fi, iota + np.int32(16 * s)], zero16)

    @pl.loop(np.int32(0), np.int32(_TROWS // _CH), step=np.int32(1))
    def _zc(b):
        pltpu.sync_copy(rows_a, acc.at[pl.ds(sid * np.int32(_TROWS) + b * np.int32(_CH), _CH)])
    plsc.subcore_barrier()

    pltpu.sync_copy(ea_t.at[wid], eabuf)
    pltpu.sync_copy(row_t.at[wid, np.int32(0)], rib.at[np.int32(0)])
    pltpu.sync_copy(col_t.at[wid, np.int32(0)], cib.at[np.int32(0)])
    pltpu.sync_copy(row_t.at[wid, np.int32(1)], rib.at[np.int32(1)])
    pltpu.sync_copy(col_t.at[wid, np.int32(1)], cib.at[np.int32(1)])
    pltpu.async_copy(u.at[rib.at[np.int32(0)]], rows_a, gsem_a)
    pltpu.async_copy(u.at[rib.at[np.int32(1)]], rows_b, gsem_b)

    @pl.loop(np.int32(0), np.int32(_NCH), step=np.int32(2))
    def _chunk2(g):
        for b in range(2):
            j = g + np.int32(b)
            fj = _splat(j)
            rows_v = bufs[b]
            jn = j + np.int32(2)
            sn = lax.rem(jn, np.int32(3))
            s0 = lax.rem(j, np.int32(3))

            pltpu.make_async_copy(u.at[rib.at[s0]], rows_v, gsems[b]).wait()

            @pl.when(jn < np.int32(_NCH))
            def _():
                pltpu.async_copy(row_t.at[wid, jn], rib.at[sn], isems[b])
                pltpu.async_copy(col_t.at[wid, jn], cib.at[sn], isems[b])

            @pl.loop(np.int32(0), np.int32(_CH), step=np.int32(1), unroll=8)
            def _edge(e):
                fe = _splat(e)
                a = -plsc.load_gather(eabuf, [fj, fe])
                for s in range(_D // 16):
                    ix = iota + np.int32(16 * s)
                    v = plsc.load_gather(rows_v, [fe, ix])
                    plsc.store_scatter(rows_v, [fe, ix], v * a)

            pltpu.sync_copy(rows_v, acc.at[cib.at[s0]], add=True)

            @pl.when(jn < np.int32(_NCH))
            def _():
                pltpu.make_async_copy(row_t.at[wid, jn], rib.at[sn], isems[b]).wait()
                pltpu.make_async_copy(col_t.at[wid, jn], cib.at[sn], isems[b]).wait()
                pltpu.async_copy(u.at[rib.at[sn]], rows_v, gsems[b])

    plsc.subcore_barrier()
    pltpu.sync_copy(acc.at[pl.ds(sid * np.int32(_TROWS), _TROWS)],
                    p_out.at[cid, pl.ds(sid * np.int32(_TROWS), _TROWS)])


_sc_spmv = pl.kernel(
    _sc_spmv_body,
    out_type=jax.ShapeDtypeStruct((_NC, _NP, _D), _f32),
    mesh=plsc.VectorSubcoreMesh(core_axis_name="c", subcore_axis_name="s"),
    compiler_params=pltpu.CompilerParams(needs_layout_passes=False),
    scratch_types=[
        pltpu.VMEM_SHARED((_NP, _D), _f32),
        pltpu.VMEM((3, _CH), _i32),
        pltpu.VMEM((3, _CH), _i32),
        pltpu.VMEM((_NCH, _CH), _f32),
        pltpu.VMEM((_CH, _D), _f32),
        pltpu.VMEM((_CH, _D), _f32),
        pltpu.SemaphoreType.DMA,
        pltpu.SemaphoreType.DMA,
        pltpu.SemaphoreType.DMA,
        pltpu.SemaphoreType.DMA,
    ],
)


def _gelu(v):
    return 0.5 * v * (1.0 + lax.erf(v * 0.7071067811865476))


def _tc_prep_body(degp_ref, x_ref, dinv_ref, u0_ref):
    d = degp_ref[0, :, 0:1] + degp_ref[1, :, 0:1]
    safe = jnp.where(d > 0, d, 1.0)
    dinv = jnp.where(d > 0, 1.0 / jnp.sqrt(safe), 0.0)
    db = jnp.broadcast_to(dinv, (_NP, _D))
    dinv_ref[...] = db
    u0_ref[...] = db * x_ref[...]


_tc_prep = pl.pallas_call(
    _tc_prep_body,
    out_shape=(jax.ShapeDtypeStruct((_NP, _D), _f32),
               jax.ShapeDtypeStruct((_NP, _D), _f32)),
)


def _tc_step1_body(p_ref, dinv_ref, tx_ref, u_ref):
    db = dinv_ref[...]
    t = db * (p_ref[0] + p_ref[1])
    tx_ref[...] = t
    u_ref[...] = db * t


_tc_step1 = pl.pallas_call(
    _tc_step1_body,
    out_shape=(jax.ShapeDtypeStruct((_NP, _D), _f32),
               jax.ShapeDtypeStruct((_NP, _D), _f32)),
)


def _tc_stepk_body(p_ref, dinv_ref, tprev_ref, tx_ref, u_ref):
    db = dinv_ref[...]
    t = 2.0 * (db * (p_ref[0] + p_ref[1])) - tprev_ref[...]
    tx_ref[...] = t
    u_ref[...] = db * t


_tc_stepk = pl.pallas_call(
    _tc_stepk_body,
    out_shape=(jax.ShapeDtypeStruct((_NP, _D), _f32),
               jax.ShapeDtypeStruct((_NP, _D), _f32)),
)


def _tc_accum1_body(v0_ref, t1_ref, t2_ref, t3_ref, t4_ref, w_ref, b_ref,
                    dinv_ref, h_ref, uh_ref):
    acc = jnp.dot(v0_ref[...], w_ref[0], preferred_element_type=_f32)
    acc += jnp.dot(t1_ref[...], w_ref[1], preferred_element_type=_f32)
    acc += jnp.dot(t2_ref[...], w_ref[2], preferred_element_type=_f32)
    acc += jnp.dot(t3_ref[...], w_ref[3], preferred_element_type=_f32)
    acc += jnp.dot(t4_ref[...], w_ref[4], preferred_element_type=_f32)
    h = _gelu(acc + b_ref[...])
    h_ref[...] = h
    uh_ref[...] = dinv_ref[...] * h


_tc_accum1 = pl.pallas_call(
    _tc_accum1_body,
    out_shape=(jax.ShapeDtypeStruct((_NP, _D), _f32),
               jax.ShapeDtypeStruct((_NP, _D), _f32)),
)


def _tc_accum2_body(v0_ref, t1_ref, t2_ref, t3_ref, t4_ref, w_ref, b_ref,
                    x_ref, wlt_ref, bl_ref, out_ref):
    acc = jnp.dot(v0_ref[...], w_ref[0], preferred_element_type=_f32)
    acc += jnp.dot(t1_ref[...], w_ref[1], preferred_element_type=_f32)
    acc += jnp.dot(t2_ref[...], w_ref[2], preferred_element_type=_f32)
    acc += jnp.dot(t3_ref[...], w_ref[3], preferred_element_type=_f32)
    acc += jnp.dot(t4_ref[...], w_ref[4], preferred_element_type=_f32)
    lin = jnp.dot(x_ref[...], wlt_ref[...], preferred_element_type=_f32)
    out_ref[...] = _gelu(acc + b_ref[...] + lin + bl_ref[...])


_tc_accum2 = pl.pallas_call(
    _tc_accum2_body,
    out_shape=jax.ShapeDtypeStruct((_NP, _D), _f32),
)


def kernel(x, edge_index, edge_attr, W1, b1, W2, b2, Wl, bl):
    with _x64_ctx(False):
        out = _kernel32(x, edge_index, edge_attr, W1, b1, W2, b2, Wl, bl)
    return out.astype(jnp.float64)


def _kernel32(x, edge_index, edge_attr, W1, b1, W2, b2, Wl, bl):
    x = x.astype(_f32)
    row = edge_index[0].astype(_i32)
    col = edge_index[1].astype(_i32)
    ea = edge_attr.astype(_f32)
    W1 = W1.astype(_f32)
    W2 = W2.astype(_f32)

    pad = _EP - _E
    row_t = jnp.concatenate([row, jnp.zeros((pad,), _i32)]).reshape(_NW, _NCH, _CH)
    col_t = jnp.concatenate([col, jnp.zeros((pad,), _i32)]).reshape(_NW, _NCH, _CH)
    ea_t = jnp.concatenate([ea, jnp.zeros((pad,), _f32)]).reshape(_NW, _NCH, _CH)
    x_pad = jnp.concatenate([x, jnp.zeros((_NP - _N, _D), _f32)], axis=0)

    b1r = b1.astype(_f32).reshape(1, _D)
    b2r = b2.astype(_f32).reshape(1, _D)
    blr = bl.astype(_f32).reshape(1, _D)
    wlt = Wl.astype(_f32).T

    degp = _sc_deg(row_t, ea_t)
    dinv_b, u0 = _tc_prep(degp, x_pad)

    def cheb_txs(v0, u_first):
        p = _sc_spmv(u_first, row_t, col_t, ea_t)
        t1, u = _tc_step1(p, dinv_b)
        p = _sc_spmv(u, row_t, col_t, ea_t)
        t2, u = _tc_stepk(p, dinv_b, v0)
        p = _sc_spmv(u, row_t, col_t, ea_t)
        t3, u = _tc_stepk(p, dinv_b, t1)
        p = _sc_spmv(u, row_t, col_t, ea_t)
        t4, _ = _tc_stepk(p, dinv_b, t2)
        return t1, t2, t3, t4

    t1, t2, t3, t4 = cheb_txs(x_pad, u0)
    h, uh = _tc_accum1(x_pad, t1, t2, t3, t4, W1, b1r, dinv_b)

    t1, t2, t3, t4 = cheb_txs(h, uh)
    out_pad = _tc_accum2(h, t1, t2, t3, t4, W2, b2r, x_pad, wlt, blr)

    return out_pad[:_N]
